# baseline (speedup 1.0000x reference)
import functools

import jax
import jax.numpy as jnp
from jax import lax
from jax.experimental import pallas as pl
from jax.experimental.pallas import tpu as pltpu
from jax.experimental.pallas import tpu_sc as plsc

N = 50000
E = 800000
H = 64
NC = 2
NS = 16
NP = 50176
HALF = NP // 2
STRIPE = HALF // NS
ES = E // NS
CE = 2000
CAP = 2288
C = 128
RB = 1024

_mesh = plsc.VectorSubcoreMesh(core_axis_name="c", subcore_axis_name="s")


def _make_spmv(W):
    @functools.partial(
        pl.kernel,
        out_type=jax.ShapeDtypeStruct((NP, W), jnp.float32),
        mesh=_mesh,
        scratch_types=[
            pltpu.VMEM((CE,), jnp.int32),
            pltpu.VMEM((CE,), jnp.int32),
            pltpu.VMEM((CE,), jnp.float32),
            pltpu.VMEM((CAP,), jnp.int32),
            pltpu.VMEM((CAP,), jnp.int32),
            pltpu.VMEM((CAP,), jnp.float32),
            pltpu.VMEM((C, W), jnp.float32),
            pltpu.VMEM((C, W), jnp.float32),
            pltpu.VMEM((C,), jnp.int32),
            pltpu.VMEM((C,), jnp.int32),
            pltpu.VMEM_SHARED((HALF, W), jnp.float32),
            pltpu.SemaphoreType.DMA,
            pltpu.SemaphoreType.DMA,
            pltpu.SemaphoreType.DMA,
            pltpu.SemaphoreType.DMA,
        ],
        compiler_params=pltpu.CompilerParams(use_tc_tiling_on_sc=False,
                                             needs_layout_passes=False),
    )
    def spmv(h_hbm, src_hbm, dst_hbm, w_hbm, z_hbm, out_hbm,
             s_stage, d_stage, w_stage, s_list, d_list, w_list,
             rows0_v, rows1_v, idx0_v, idx1_v, acc_sh,
             sem_st, sem_g0, sem_g1, sem_s0):
        cid = lax.axis_index("c")
        sid = lax.axis_index("s")
        base = cid * HALF
        rows = (rows0_v, rows1_v)
        idxs = (idx0_v, idx1_v)
        sem_g = (sem_g0, sem_g1)

        pltpu.sync_copy(z_hbm.at[pl.ds(base + sid * STRIPE, STRIPE)],
                        acc_sh.at[pl.ds(sid * STRIPE, STRIPE)])

        zi16 = jnp.zeros((16,), jnp.int32)
        zf16 = jnp.zeros((16,), jnp.float32)

        def zlist(g, carry):
            s_list[pl.ds(g * 16, 16)] = zi16
            return carry

        lax.fori_loop(0, CAP // 16, zlist, 0)

        plsc.subcore_barrier()

        sstart = sid * ES
        trash = jnp.int32(CAP - 16) + lax.iota(jnp.int32, 16)

        def scale(buf, jb):
            def srow(r4, rcarry):
                for u in range(4):
                    r = r4 * 4 + u
                    wv = w_list[pl.ds(jb + r, 16)][0]
                    for q in range(W // 16):
                        sl = pl.ds(16 * q, 16)
                        buf[r, sl] = buf[r, sl] * wv
                return rcarry

            lax.fori_loop(0, C // 4, srow, 0)

        def staged(k, scarry):
            koff = pl.multiple_of(sstart + k * CE, 8)
            cst = pltpu.async_copy(src_hbm.at[pl.ds(koff, CE)], s_stage,
                                   sem_st)
            cdt = pltpu.async_copy(dst_hbm.at[pl.ds(koff, CE)], d_stage,
                                   sem_st)
            cwt = pltpu.async_copy(w_hbm.at[pl.ds(koff, CE)], w_stage,
                                   sem_st)
            cst.wait()
            cdt.wait()
            cwt.wait()

            def group(g, cnt):
                off = g * 16
                d16 = d_stage[pl.ds(off, 16)]
                s16 = s_stage[pl.ds(off, 16)]
                w16 = w_stage[pl.ds(off, 16)]
                dl = d16 - base
                m = (dl >= 0) & (dl < HALF)
                cs = plsc.cumsum(jnp.where(m, jnp.int32(1), jnp.int32(0)))
                pos = jnp.where(m, cnt + cs - 1, trash)
                plsc.store_scatter(s_list, [pos], s16)
                plsc.store_scatter(d_list, [pos], jnp.where(m, dl, 0))
                plsc.store_scatter(w_list, [pos], w16)
                return cnt + cs[15]

            cnt = lax.fori_loop(0, CE // 16, group, jnp.int32(0))

            for t in range(2 * C // 16):
                s_list[pl.ds(cnt + 16 * t, 16)] = zi16
                d_list[pl.ds(cnt + 16 * t, 16)] = zi16
                w_list[pl.ds(cnt + 16 * t, 16)] = zf16

            nch2 = (cnt + (2 * C - 1)) // (2 * C)

            pltpu.async_copy(h_hbm.at[s_list.at[pl.ds(0, C)]], rows[0],
                             sem_g[0])

            def pair(j2, carry):
                jb = pl.multiple_of(j2 * (2 * C), 8)
                g1 = pltpu.async_copy(h_hbm.at[s_list.at[pl.ds(jb + C, C)]],
                                      rows[1], sem_g[1])
                pltpu.make_async_copy(h_hbm.at[s_list.at[pl.ds(jb, C)]],
                                      rows[0], sem_g[0]).wait()
                scale(rows[0], jb)
                for t in range(C // 16):
                    idxs[0][pl.ds(16 * t, 16)] = d_list[pl.ds(jb + 16 * t, 16)]
                s0 = pltpu.async_copy(rows[0], acc_sh.at[idxs[0]], sem_s0,
                                      add=True)
                g1.wait()
                scale(rows[1], jb + C)
                for t in range(C // 16):
                    idxs[1][pl.ds(16 * t, 16)] = d_list[
                        pl.ds(jb + C + 16 * t, 16)]
                pltpu.sync_copy(rows[1], acc_sh.at[idxs[1]], add=True)
                s0.wait()
                pltpu.async_copy(
                    h_hbm.at[s_list.at[pl.ds(jb + 2 * C, C)]], rows[0],
                    sem_g[0])
                return carry

            lax.fori_loop(0, nch2, pair, 0)
            pltpu.make_async_copy(h_hbm.at[s_list.at[pl.ds(0, C)]], rows[0],
                                  sem_g[0]).wait()
            return scarry

        lax.fori_loop(0, ES // CE, staged, 0)

        plsc.subcore_barrier()
        pltpu.sync_copy(acc_sh.at[pl.ds(sid * STRIPE, STRIPE)],
                        out_hbm.at[pl.ds(base + sid * STRIPE, STRIPE)])

    return spmv


_spmv64 = _make_spmv(H)
_spmv16 = _make_spmv(16)


def _dense_mid_body(agg_ref, h_ref, wr_ref, wroot_ref, b_ref, o_ref):
    acc = jnp.dot(agg_ref[...], wr_ref[...], preferred_element_type=jnp.float32)
    acc = acc + jnp.dot(h_ref[...], wroot_ref[...],
                        preferred_element_type=jnp.float32)
    acc = acc + b_ref[...]
    o_ref[...] = jnp.maximum(acc, 0.0)


_dense_mid = pl.pallas_call(
    _dense_mid_body,
    grid=(NP // RB,),
    in_specs=[
        pl.BlockSpec((RB, H), lambda i: (i, 0)),
        pl.BlockSpec((RB, H), lambda i: (i, 0)),
        pl.BlockSpec((H, H), lambda i: (0, 0)),
        pl.BlockSpec((H, H), lambda i: (0, 0)),
        pl.BlockSpec((1, H), lambda i: (0, 0)),
    ],
    out_specs=pl.BlockSpec((RB, H), lambda i: (i, 0)),
    out_shape=jax.ShapeDtypeStruct((NP, H), jnp.float32),
)


def _dense_first_body(u_ref, x_ref, wr_ref, wroot_ref, b_ref, o_ref):
    u = u_ref[...][:, 0][:, None]
    x = x_ref[...][:, 0][:, None]
    acc = u * wr_ref[...] + x * wroot_ref[...] + b_ref[...]
    o_ref[...] = jnp.maximum(acc, 0.0)


_dense_first = pl.pallas_call(
    _dense_first_body,
    grid=(NP // RB,),
    in_specs=[
        pl.BlockSpec((RB, 16), lambda i: (i, 0)),
        pl.BlockSpec((RB, 16), lambda i: (i, 0)),
        pl.BlockSpec((1, H), lambda i: (0, 0)),
        pl.BlockSpec((1, H), lambda i: (0, 0)),
        pl.BlockSpec((1, H), lambda i: (0, 0)),
    ],
    out_specs=pl.BlockSpec((RB, H), lambda i: (i, 0)),
    out_shape=jax.ShapeDtypeStruct((NP, H), jnp.float32),
)


def _pre_last_body(h_ref, wr_ref, wroot_ref, b_ref, y_ref, z_ref):
    h = h_ref[...]
    y = jnp.sum(h * wr_ref[...], axis=1)
    z = jnp.sum(h * wroot_ref[...], axis=1) + b_ref[0, 0]
    y_ref[...] = jnp.broadcast_to(y[:, None], (RB, 16))
    z_ref[...] = jnp.broadcast_to(z[:, None], (RB, 16))


_pre_last = pl.pallas_call(
    _pre_last_body,
    grid=(NP // RB,),
    in_specs=[
        pl.BlockSpec((RB, H), lambda i: (i, 0)),
        pl.BlockSpec((1, H), lambda i: (0, 0)),
        pl.BlockSpec((1, H), lambda i: (0, 0)),
        pl.BlockSpec((1, 1), lambda i: (0, 0)),
    ],
    out_specs=[
        pl.BlockSpec((RB, 16), lambda i: (i, 0)),
        pl.BlockSpec((RB, 16), lambda i: (i, 0)),
    ],
    out_shape=[
        jax.ShapeDtypeStruct((NP, 16), jnp.float32),
        jax.ShapeDtypeStruct((NP, 16), jnp.float32),
    ],
)


def kernel(x, edge_index, edge_weights,
           W_rel0, b_rel0, W_root0,
           W_rel1, b_rel1, W_root1,
           W_rel2, b_rel2, W_root2,
           W_rel3, b_rel3, W_root3,
           W_rel4, b_rel4, W_root4):
    src = edge_index[0]
    dst = edge_index[1]
    w = edge_weights

    zeros16 = jnp.zeros((NP, 16), jnp.float32)
    zeros64 = jnp.zeros((NP, H), jnp.float32)

    x16 = jnp.broadcast_to(
        jnp.concatenate([x[:, 0], jnp.zeros((NP - N,), jnp.float32)])[:, None],
        (NP, 16))
    u = _spmv16(x16, src, dst, w, zeros16)
    h = _dense_first(u, x16, W_rel0[:, 0][None, :], W_root0[:, 0][None, :],
                     b_rel0[None, :])

    for W_rel, b_rel, W_root in ((W_rel1, b_rel1, W_root1),
                                 (W_rel2, b_rel2, W_root2),
                                 (W_rel3, b_rel3, W_root3)):
        agg = _spmv64(h, src, dst, w, zeros64)
        h = _dense_mid(agg, h, W_rel.T, W_root.T, b_rel[None, :])

    y, z = _pre_last(h, W_rel4, W_root4, b_rel4.reshape(1, 1))
    out = _spmv16(y, src, dst, w, z)
    return out[:N, 0][:, None]


# speedup vs baseline: 2.4716x; 2.4716x over previous
import functools

import jax
import jax.numpy as jnp
from jax import lax
from jax.experimental import pallas as pl
from jax.experimental.pallas import tpu as pltpu
from jax.experimental.pallas import tpu_sc as plsc

N = 50000
E = 800000
H = 64
NC = 2
NS = 16
NP = 51200
HALF = NP // 2
STRIPE = HALF // NS
ES = E // NS
CE = 2000
CAP = 2160
C = 128
RB = 1024

_mesh = plsc.VectorSubcoreMesh(core_axis_name="c", subcore_axis_name="s")


def _make_spmv(W):
    @functools.partial(
        pl.kernel,
        out_type=jax.ShapeDtypeStruct((NP, W), jnp.float32),
        mesh=_mesh,
        scratch_types=[
            pltpu.VMEM((CE,), jnp.int32),
            pltpu.VMEM((CE,), jnp.int32),
            pltpu.VMEM((CE,), jnp.float32),
            pltpu.VMEM((CAP,), jnp.int32),
            pltpu.VMEM((CAP,), jnp.int32),
            pltpu.VMEM((CAP,), jnp.float32),
            pltpu.VMEM((C, W), jnp.float32),
            pltpu.VMEM((C,), jnp.int32),
            pltpu.VMEM_SHARED((HALF, W), jnp.float32),
            pltpu.SemaphoreType.DMA,
        ],
        compiler_params=pltpu.CompilerParams(use_tc_tiling_on_sc=False,
                                             needs_layout_passes=False),
    )
    def spmv(h_hbm, src_hbm, dst_hbm, w_hbm, z_hbm, out_hbm,
             s_stage, d_stage, w_stage, s_list, d_list, w_list,
             rows_v, idx_v, acc_sh, sem):
        cid = lax.axis_index("c")
        sid = lax.axis_index("s")
        base = cid * HALF

        pltpu.sync_copy(z_hbm.at[pl.ds(base + sid * STRIPE, STRIPE)],
                        acc_sh.at[pl.ds(sid * STRIPE, STRIPE)])

        plsc.subcore_barrier()

        sstart = sid * ES
        trash = jnp.int32(CAP - 16) + lax.iota(jnp.int32, 16)
        zi16 = jnp.zeros((16,), jnp.int32)
        zf16 = jnp.zeros((16,), jnp.float32)

        for k in range(ES // CE):
            koff = sstart + k * CE
            c1 = pltpu.async_copy(src_hbm.at[pl.ds(koff, CE)], s_stage, sem)
            c2 = pltpu.async_copy(dst_hbm.at[pl.ds(koff, CE)], d_stage, sem)
            c3 = pltpu.async_copy(w_hbm.at[pl.ds(koff, CE)], w_stage, sem)
            c1.wait()
            c2.wait()
            c3.wait()

            def group(g, cnt):
                off = g * 16
                d16 = d_stage[pl.ds(off, 16)]
                s16 = s_stage[pl.ds(off, 16)]
                w16 = w_stage[pl.ds(off, 16)]
                dl = d16 - base
                m = (dl >= 0) & (dl < HALF)
                cs = plsc.cumsum(jnp.where(m, jnp.int32(1), jnp.int32(0)))
                pos = jnp.where(m, cnt + cs - 1, trash)
                plsc.store_scatter(s_list, [pos], s16)
                plsc.store_scatter(d_list, [pos], jnp.where(m, dl, 0))
                plsc.store_scatter(w_list, [pos], w16)
                return cnt + cs[15]

            cnt = lax.fori_loop(0, CE // 16, group, jnp.int32(0))

            for t in range(C // 16):
                s_list[pl.ds(cnt + 16 * t, 16)] = zi16
                d_list[pl.ds(cnt + 16 * t, 16)] = zi16
                w_list[pl.ds(cnt + 16 * t, 16)] = zf16

            nch = (cnt + (C - 1)) // C

            def chunk(j, carry):
                jb = pl.multiple_of(j * C, 8)
                pltpu.async_copy(h_hbm.at[s_list.at[pl.ds(jb, C)]], rows_v,
                                 sem).wait()

                def srow(r4, rcarry):
                    for u in range(4):
                        r = r4 * 4 + u
                        wv = w_list[pl.ds(jb + r, 16)][0]
                        for q in range(W // 16):
                            sl = pl.ds(16 * q, 16)
                            rows_v[r, sl] = rows_v[r, sl] * wv
                    return rcarry

                lax.fori_loop(0, C // 4, srow, 0)
                for t in range(C // 16):
                    idx_v[pl.ds(16 * t, 16)] = d_list[pl.ds(jb + 16 * t, 16)]
                pltpu.sync_copy(rows_v, acc_sh.at[idx_v], add=True)
                return carry

            lax.fori_loop(0, nch, chunk, 0)

        plsc.subcore_barrier()
        pltpu.sync_copy(acc_sh.at[pl.ds(sid * STRIPE, STRIPE)],
                        out_hbm.at[pl.ds(base + sid * STRIPE, STRIPE)])

    return spmv


_spmv64 = _make_spmv(H)
_spmv16 = _make_spmv(16)


def _dense_mid_body(agg_ref, h_ref, wr_ref, wroot_ref, b_ref, o_ref):
    acc = jnp.dot(agg_ref[...], wr_ref[...], preferred_element_type=jnp.float32)
    acc = acc + jnp.dot(h_ref[...], wroot_ref[...],
                        preferred_element_type=jnp.float32)
    acc = acc + b_ref[...]
    o_ref[...] = jnp.maximum(acc, 0.0)


_dense_mid = pl.pallas_call(
    _dense_mid_body,
    grid=(NP // RB,),
    in_specs=[
        pl.BlockSpec((RB, H), lambda i: (i, 0)),
        pl.BlockSpec((RB, H), lambda i: (i, 0)),
        pl.BlockSpec((H, H), lambda i: (0, 0)),
        pl.BlockSpec((H, H), lambda i: (0, 0)),
        pl.BlockSpec((1, H), lambda i: (0, 0)),
    ],
    out_specs=pl.BlockSpec((RB, H), lambda i: (i, 0)),
    out_shape=jax.ShapeDtypeStruct((NP, H), jnp.float32),
)


def _dense_first_body(u_ref, x_ref, wr_ref, wroot_ref, b_ref, o_ref):
    u = u_ref[...][:, 0][:, None]
    x = x_ref[...][:, 0][:, None]
    acc = u * wr_ref[...] + x * wroot_ref[...] + b_ref[...]
    o_ref[...] = jnp.maximum(acc, 0.0)


_dense_first = pl.pallas_call(
    _dense_first_body,
    grid=(NP // RB,),
    in_specs=[
        pl.BlockSpec((RB, 16), lambda i: (i, 0)),
        pl.BlockSpec((RB, 16), lambda i: (i, 0)),
        pl.BlockSpec((1, H), lambda i: (0, 0)),
        pl.BlockSpec((1, H), lambda i: (0, 0)),
        pl.BlockSpec((1, H), lambda i: (0, 0)),
    ],
    out_specs=pl.BlockSpec((RB, H), lambda i: (i, 0)),
    out_shape=jax.ShapeDtypeStruct((NP, H), jnp.float32),
)


def _pre_last_body(h_ref, wr_ref, wroot_ref, b_ref, y_ref, z_ref):
    h = h_ref[...]
    y = jnp.sum(h * wr_ref[...], axis=1)
    z = jnp.sum(h * wroot_ref[...], axis=1) + b_ref[0, 0]
    y_ref[...] = jnp.broadcast_to(y[:, None], (RB, 16))
    z_ref[...] = jnp.broadcast_to(z[:, None], (RB, 16))


_pre_last = pl.pallas_call(
    _pre_last_body,
    grid=(NP // RB,),
    in_specs=[
        pl.BlockSpec((RB, H), lambda i: (i, 0)),
        pl.BlockSpec((1, H), lambda i: (0, 0)),
        pl.BlockSpec((1, H), lambda i: (0, 0)),
        pl.BlockSpec((1, 1), lambda i: (0, 0)),
    ],
    out_specs=[
        pl.BlockSpec((RB, 16), lambda i: (i, 0)),
        pl.BlockSpec((RB, 16), lambda i: (i, 0)),
    ],
    out_shape=[
        jax.ShapeDtypeStruct((NP, 16), jnp.float32),
        jax.ShapeDtypeStruct((NP, 16), jnp.float32),
    ],
)


def kernel(x, edge_index, edge_weights,
           W_rel0, b_rel0, W_root0,
           W_rel1, b_rel1, W_root1,
           W_rel2, b_rel2, W_root2,
           W_rel3, b_rel3, W_root3,
           W_rel4, b_rel4, W_root4):
    src = edge_index[0]
    dst = edge_index[1]
    w = edge_weights

    zeros16 = jnp.zeros((NP, 16), jnp.float32)
    zeros64 = jnp.zeros((NP, H), jnp.float32)

    x16 = jnp.broadcast_to(
        jnp.concatenate([x[:, 0], jnp.zeros((NP - N,), jnp.float32)])[:, None],
        (NP, 16))
    u = _spmv16(x16, src, dst, w, zeros16)
    h = _dense_first(u, x16, W_rel0[:, 0][None, :], W_root0[:, 0][None, :],
                     b_rel0[None, :])

    for W_rel, b_rel, W_root in ((W_rel1, b_rel1, W_root1),
                                 (W_rel2, b_rel2, W_root2),
                                 (W_rel3, b_rel3, W_root3)):
        agg = _spmv64(h, src, dst, w, zeros64)
        h = _dense_mid(agg, h, W_rel.T, W_root.T, b_rel[None, :])

    y, z = _pre_last(h, W_rel4, W_root4, b_rel4.reshape(1, 1))
    out = _spmv16(y, src, dst, w, z)
    return out[:N, 0][:, None]
